# Initial kernel scaffold; baseline (speedup 1.0000x reference)
import jax
import jax.numpy as jnp
from jax.experimental import pallas as pl


def kernel(z, edge_index):
    raise NotImplementedError("write your pallas kernel here")


# speedup vs baseline: 3.6748x; 3.6748x over previous
import jax
import jax.numpy as jnp
from jax import lax
from jax.experimental import pallas as pl
from jax.experimental.pallas import tpu as pltpu
from jax.experimental.pallas import tpu_sc as plsc

N_NODES = 10000
D_FEAT = 128
N_EDGES = 320000
EPS = 1e-6

NC = 2
NS = 16
NW = NC * NS
E_W = N_EDGES // NW
B = 80
NCHUNK = E_W // B
GROUPS = B // 16


def _dist_body(z_hbm, src_hbm, dst_hbm, out_hbm,
               src_idx, dst_idx, src_rows, dst_rows, out_v, sem):
    wid = lax.axis_index("s") * NC + lax.axis_index("c")
    base = wid * E_W

    def chunk_body(j, carry):
        off = base + j * B
        pltpu.sync_copy(src_hbm.at[pl.ds(off, B)], src_idx)
        pltpu.sync_copy(dst_hbm.at[pl.ds(off, B)], dst_idx)
        c1 = pltpu.async_copy(z_hbm.at[src_idx], src_rows, sem)
        c2 = pltpu.async_copy(z_hbm.at[dst_idx], dst_rows, sem)
        c1.wait()
        c2.wait()

        lanes = lax.iota(jnp.int32, 16)

        perms = [(lanes + sh) & 15 for sh in (8, 4, 2, 1)]

        def group_body(g, carry2):
            res = jnp.zeros((16,), jnp.float32)
            for e in range(16):
                row = g * 16 + e
                acc0 = jnp.zeros((16,), jnp.float32)
                acc1 = jnp.zeros((16,), jnp.float32)
                for k in range(8):
                    sv = src_rows[row, pl.ds(k * 16, 16)]
                    dv = dst_rows[row, pl.ds(k * 16, 16)]
                    dif = sv - dv + EPS
                    if k % 2 == 0:
                        acc0 = acc0 + dif * dif
                    else:
                        acc1 = acc1 + dif * dif
                acc = acc0 + acc1
                for p in perms:
                    acc = acc + lax.gather(
                        acc, p[:, None],
                        dimension_numbers=lax.GatherDimensionNumbers(
                            offset_dims=(), collapsed_slice_dims=(0,),
                            start_index_map=(0,)),
                        slice_sizes=(1,),
                        mode=lax.GatherScatterMode.PROMISE_IN_BOUNDS)
                res = jnp.where(lanes == e, acc, res)
            ibits = lax.bitcast_convert_type(res, jnp.int32)
            y = lax.bitcast_convert_type(
                jnp.int32(0x5F3759DF) - (ibits >> 1), jnp.float32)
            for _ in range(3):
                y = y * (1.5 - 0.5 * res * y * y)
            dist = res * y
            ex = jnp.exp(-dist)
            out_v[pl.ds(g * 16, 16)] = ex / (1.0 + ex)
            return carry2

        lax.fori_loop(0, GROUPS, group_body, 0)
        pltpu.sync_copy(out_v, out_hbm.at[pl.ds(off, B)])
        return carry

    lax.fori_loop(0, NCHUNK, chunk_body, 0)


def kernel(z, edge_index):
    src = edge_index[0].astype(jnp.int32)
    dst = edge_index[1].astype(jnp.int32)
    mesh = plsc.VectorSubcoreMesh(core_axis_name="c", subcore_axis_name="s")
    f = pl.kernel(
        _dist_body,
        mesh=mesh,
        out_type=jax.ShapeDtypeStruct((N_EDGES,), jnp.float32),
        scratch_types=[
            pltpu.VMEM((B,), jnp.int32),
            pltpu.VMEM((B,), jnp.int32),
            pltpu.VMEM((B, D_FEAT), jnp.float32),
            pltpu.VMEM((B, D_FEAT), jnp.float32),
            pltpu.VMEM((B,), jnp.float32),
            pltpu.SemaphoreType.DMA,
        ],
    )
    return f(z, src, dst)


# speedup vs baseline: 4.6245x; 1.2584x over previous
import jax
import jax.numpy as jnp
from jax import lax
from jax.experimental import pallas as pl
from jax.experimental.pallas import tpu as pltpu
from jax.experimental.pallas import tpu_sc as plsc

N_NODES = 10000
D_FEAT = 128
N_EDGES = 320000
EPS = 1e-6

NC = 2
NS = 16
NW = NC * NS
E_W = N_EDGES // NW
B = 80
NCHUNK = E_W // B
NPAIR = (NCHUNK - 1) // 2


def _lane_gather(x, perm):
    return lax.gather(
        x, perm[:, None],
        dimension_numbers=lax.GatherDimensionNumbers(
            offset_dims=(), collapsed_slice_dims=(0,), start_index_map=(0,)),
        slice_sizes=(1,),
        mode=lax.GatherScatterMode.PROMISE_IN_BOUNDS)


def _dist_body(z_hbm, src_hbm, dst_hbm, out_hbm,
               src_idx, dst_idx, src_r0, dst_r0, src_r1, dst_r1,
               out_v, sem0, sem1):
    wid = lax.axis_index("s") * NC + lax.axis_index("c")
    base = wid * E_W

    pltpu.sync_copy(src_hbm.at[wid], src_idx)
    pltpu.sync_copy(dst_hbm.at[wid], dst_idx)

    def issue(j, rows_s, rows_d, sem):
        pltpu.async_copy(z_hbm.at[src_idx.at[j]], rows_s, sem)
        pltpu.async_copy(z_hbm.at[dst_idx.at[j]], rows_d, sem)

    def wait(j, rows_s, rows_d, sem):
        pltpu.make_async_copy(z_hbm.at[src_idx.at[j]], rows_s, sem).wait()
        pltpu.make_async_copy(z_hbm.at[dst_idx.at[j]], rows_d, sem).wait()

    lanes = lax.iota(jnp.int32, 16)
    perms = {sh: lanes ^ sh for sh in (1, 2, 4, 8)}
    masks = {sh: (lanes & sh) == 0 for sh in (1, 2, 4, 8)}

    def merge(x, y, sh):
        return jnp.where(masks[sh],
                         x + _lane_gather(x, perms[sh]),
                         y + _lane_gather(y, perms[sh]))

    def compute(j, rows_s, rows_d):
        for g in range(B // 16):
            vecs = []
            for e in range(16):
                row = g * 16 + e
                acc0 = jnp.zeros((16,), jnp.float32)
                acc1 = jnp.zeros((16,), jnp.float32)
                for k in range(8):
                    sv = rows_s[row, pl.ds(k * 16, 16)]
                    dv = rows_d[row, pl.ds(k * 16, 16)]
                    dif = sv - dv + EPS
                    if k % 2 == 0:
                        acc0 = acc0 + dif * dif
                    else:
                        acc1 = acc1 + dif * dif
                vecs.append(acc0 + acc1)
            l8 = [merge(vecs[e], vecs[e + 8], 8) for e in range(8)]
            l4 = [merge(l8[e], l8[e + 4], 4) for e in range(4)]
            l2 = [merge(l4[e], l4[e + 2], 2) for e in range(2)]
            res = merge(l2[0], l2[1], 1)
            ibits = lax.bitcast_convert_type(res, jnp.int32)
            y = lax.bitcast_convert_type(
                jnp.int32(0x5F3759DF) - (ibits >> 1), jnp.float32)
            for _ in range(3):
                y = y * (1.5 - 0.5 * res * y * y)
            dist = res * y
            ex = jnp.exp(-dist)
            out_v[pl.ds(g * 16, 16)] = ex / (1.0 + ex)
        pltpu.sync_copy(out_v, out_hbm.at[pl.ds(base + j * B, B)])

    issue(0, src_r0, dst_r0, sem0)

    def pair_body(t, carry):
        j0 = 2 * t
        issue(j0 + 1, src_r1, dst_r1, sem1)
        wait(j0, src_r0, dst_r0, sem0)
        compute(j0, src_r0, dst_r0)
        issue(j0 + 2, src_r0, dst_r0, sem0)
        wait(j0 + 1, src_r1, dst_r1, sem1)
        compute(j0 + 1, src_r1, dst_r1)
        return carry

    lax.fori_loop(0, NPAIR, pair_body, 0)
    wait(NCHUNK - 1, src_r0, dst_r0, sem0)
    compute(NCHUNK - 1, src_r0, dst_r0)


def kernel(z, edge_index):
    src = edge_index[0].astype(jnp.int32).reshape(NW, NCHUNK, B)
    dst = edge_index[1].astype(jnp.int32).reshape(NW, NCHUNK, B)
    mesh = plsc.VectorSubcoreMesh(core_axis_name="c", subcore_axis_name="s")
    f = pl.kernel(
        _dist_body,
        mesh=mesh,
        out_type=jax.ShapeDtypeStruct((N_EDGES,), jnp.float32),
        scratch_types=[
            pltpu.VMEM((NCHUNK, B), jnp.int32),
            pltpu.VMEM((NCHUNK, B), jnp.int32),
            pltpu.VMEM((B, D_FEAT), jnp.float32),
            pltpu.VMEM((B, D_FEAT), jnp.float32),
            pltpu.VMEM((B, D_FEAT), jnp.float32),
            pltpu.VMEM((B, D_FEAT), jnp.float32),
            pltpu.VMEM((B,), jnp.float32),
            pltpu.SemaphoreType.DMA,
            pltpu.SemaphoreType.DMA,
        ],
    )
    return f(z, src, dst)
